# baseline (speedup 1.0000x reference)
import functools

import numpy as np

import jax
import jax.numpy as jnp
from jax import lax
from jax.experimental import pallas as pl
from jax.experimental.pallas import tpu as pltpu
from jax.experimental.pallas import tpu_sc as plsc

_BATCH = 16384
_NP = 20
_D = 32
_B = _BATCH * _NP
_NC = 2
_NS = 16
_NW = _NC * _NS
_BPW = _B // _NW
_GROW = 128
_GPC = 5
_CROW = _GROW * _GPC
_CB = _CROW // _NP
_NCHUNK = _BPW // _CROW
_BPWB = _BPW // _NP

_TB = 128
_TD = 8
_NTI = _D // _TD
_NTJ = _BATCH // _TB
_NQ = _NP * _D // 128

_mesh = plsc.VectorSubcoreMesh(
    core_axis_name="c", subcore_axis_name="s",
    num_cores=_NC, num_subcores=_NS)


@functools.partial(
    pl.kernel,
    out_type=jax.ShapeDtypeStruct((_BATCH, _NP, _D), jnp.float32),
    mesh=_mesh,
    compiler_params=pltpu.CompilerParams(use_tc_tiling_on_sc=False),
    scratch_types=[
        pltpu.VMEM((_BPW,), jnp.int32),
        pltpu.VMEM((_CROW, _D), jnp.float32),
        pltpu.VMEM((_CROW, _D), jnp.float32),
        pltpu.SemaphoreType.DMA,
        pltpu.SemaphoreType.DMA,
        pltpu.SemaphoreType.DMA,
        pltpu.SemaphoreType.DMA,
    ],
)
def _gather_kernel(idx_hbm, table_hbm, out_hbm, idx_v, rows0, rows1,
                   gsem0, gsem1, fsem0, fsem1):
    wid = lax.axis_index("s") * _NC + lax.axis_index("c")
    bbase = wid * _BPWB
    pltpu.sync_copy(idx_hbm.at[pl.ds(wid * _BPW, _BPW)], idx_v)

    def fire_gather(k, rows, sem):
        for g in range(_GPC):
            pltpu.async_copy(
                table_hbm.at[idx_v.at[pl.ds(k * _CROW + g * _GROW, _GROW)]],
                rows.at[pl.ds(g * _GROW, _GROW)], sem)

    def drain_gather(k, rows, sem):
        for g in range(_GPC):
            pltpu.make_async_copy(
                table_hbm.at[idx_v.at[pl.ds(k * _CROW + g * _GROW, _GROW)]],
                rows.at[pl.ds(g * _GROW, _GROW)], sem).wait()

    def fire_flush(k, rows, sem):
        for e in range(_CB):
            pltpu.async_copy(rows.at[pl.ds(e * _NP, _NP)],
                             out_hbm.at[bbase + k * _CB + e], sem)

    def drain_flush(k, rows, sem):
        for e in range(_CB):
            pltpu.make_async_copy(rows.at[pl.ds(e * _NP, _NP)],
                                  out_hbm.at[bbase + k * _CB + e],
                                  sem).wait()

    fire_gather(0, rows0, gsem0)

    @pl.loop(0, _NCHUNK, step=2)
    def _body(k):
        fire_gather(k + 1, rows1, gsem1)
        drain_gather(k, rows0, gsem0)
        fire_flush(k, rows0, fsem0)
        drain_gather(k + 1, rows1, gsem1)
        fire_flush(k + 1, rows1, fsem1)
        drain_flush(k, rows0, fsem0)

        @pl.when(k + 2 < _NCHUNK)
        def _():
            fire_gather(k + 2, rows0, gsem0)

        drain_flush(k + 1, rows1, fsem1)


def _transpose_body(in_ref, sel_ref, out_ref):
    x = in_ref[...]
    for j in range(_NQ):
        z = lax.dot_general(x, sel_ref[j], (((0,), (0,)), ((), ())),
                            precision=lax.Precision.HIGHEST,
                            preferred_element_type=jnp.float32)
        out_ref[j * 4:j * 4 + 4] = z.reshape(4, _NTI, 1, _TD, _TB)


_transpose_kernel = pl.pallas_call(
    _transpose_body,
    grid=(_NTJ,),
    in_specs=[pl.BlockSpec((_CROW, 128), lambda tj: (tj, 0)),
              pl.BlockSpec((_NQ, _CROW, _TB), lambda tj: (0, 0, 0))],
    out_specs=pl.BlockSpec((_NP, _NTI, 1, _TD, _TB),
                           lambda tj: (0, 0, tj, 0, 0)),
    out_shape=jax.ShapeDtypeStruct((_NP, _NTI, _NTJ, _TD, _TB),
                                   jnp.float32),
)

_i = np.arange(_CROW)[:, None]
_b = np.arange(_TB)[None, :]
_SEL = np.stack([(_i == _b * _NQ + j) for j in range(_NQ)]
                ).astype(np.float32)


def kernel(species_ids, table):
    flat3 = _gather_kernel(species_ids.reshape(-1).astype(jnp.int32), table)
    in2d = flat3.reshape(_B * _D // 128, 128)
    out5 = _transpose_kernel(in2d, jnp.asarray(_SEL))
    return out5.transpose(2, 4, 0, 1, 3).reshape(_BATCH, _NP, _D)
